# baseline (speedup 1.0000x reference)
import functools

import jax
import jax.numpy as jnp
from jax import lax
from jax.experimental import pallas as pl
from jax.experimental.pallas import tpu as pltpu
from jax.experimental.pallas import tpu_sc as plsc

_T, _N, _C, _PS = 2, 65536, 3, 7
_H, _W = 256, 256
_HW = _H * _W
_SW = _W - _PS + 1
_PP = _PS * _PS
_ROW = _C * _PP
_NCOMBO = _T * _C
_SUBS = 5
_NWORK = _NCOMBO * _SUBS
_CHUNK = 256
_NCHUNKS = _N // _CHUNK
_C0 = (0, 52, 103, 154, 205)
_QB = 52 * _CHUNK
_UNROLL = 8


def _sc_body(patches_hbm, qstart_hbm, part_hbm, canvas, pbufa, pbufb, pbufc,
             qbuf, sema, semb, semc):
    cid = lax.axis_index("c")
    sid = lax.axis_index("s")
    wid = sid * 2 + cid

    zero16 = jnp.zeros((16,), jnp.float32)

    @plsc.parallel_loop(0, _HW, step=16, unroll=8)
    def _zero(i):
        canvas[pl.ds(i, 16)] = zero16

    @pl.when(wid < _NWORK)
    def _work():
        plane = wid // _SUBS
        sub = wid % _SUBS
        t = plane // _C
        ch = plane % _C

        iota = lax.iota(jnp.int32, 16)
        seven = jnp.full((16,), 7, jnp.int32)
        offv = []
        for k in range(4):
            o = iota + (k * 16)
            offv.append(lax.div(o, seven) * _W + lax.rem(o, seven))
        m3 = (iota + 48) < _PP

        c0 = jnp.int32(0)
        for s in range(1, _SUBS):
            c0 = jnp.where(sub == s, jnp.int32(_C0[s]), c0)
        c1 = jnp.int32(_NCHUNKS)
        for s in range(_SUBS - 1):
            c1 = jnp.where(sub == s, jnp.int32(_C0[s + 1]), c1)

        s0 = jnp.minimum(c0 * _CHUNK, jnp.int32(_N - _QB))
        pltpu.sync_copy(qstart_hbm.at[pl.ds(s0, _QB)], qbuf)

        @plsc.parallel_loop(0, _QB, step=16, unroll=8)
        def _bases(i):
            qv = qbuf[pl.ds(i, 16)]
            qu = qv.astype(jnp.uint32)
            hi = ((qu * jnp.uint32(67109)) >> jnp.uint32(24)).astype(jnp.int32)
            qbuf[pl.ds(i, 16)] = qv + 6 * hi

        def dma(cix, buf, sem):
            return pltpu.make_async_copy(
                patches_hbm.at[
                    t, pl.ds(ch * _PP, _PP), 0, pl.ds(cix * _CHUNK, _CHUNK)
                ],
                buf,
                sem,
            )

        def compute(cix, buf):
            base_off = cix * _CHUNK - s0

            @plsc.parallel_loop(0, _CHUNK, step=1, unroll=_UNROLL)
            def _patches(j):
                jv = jnp.full((16,), j, jnp.int32)
                bspl = plsc.load_gather(qbuf, [jv + base_off])
                for k in range(3):
                    vals = plsc.load_gather(buf, [iota + (k * 16), jv])
                    plsc.addupdate_scatter(canvas, [bspl + offv[k]], vals)
                vals = plsc.load_gather(buf, [iota + 48, jv], mask=m3)
                plsc.addupdate_scatter(canvas, [bspl + offv[3]], vals, mask=m3)

        bufs = (pbufa, pbufb, pbufc)
        sems = (sema, semb, semc)

        dma(c0, pbufa, sema).start()

        @pl.when(c0 + 1 < c1)
        def _():
            dma(c0 + 1, pbufb, semb).start()

        def tri_body(i3, carry):
            base = c0 + 3 * i3
            for p in range(3):
                cp = base + p
                buf, sem = bufs[p], sems[p]
                nxt = bufs[(p + 2) % 3], sems[(p + 2) % 3]

                def phase(cp=cp, buf=buf, sem=sem, nxt=nxt):
                    dma(cp, buf, sem).wait()

                    @pl.when(cp + 2 < c1)
                    def _():
                        dma(cp + 2, nxt[0], nxt[1]).start()

                    compute(cp, buf)

                if p == 0:
                    phase()
                else:
                    pl.when(cp < c1)(phase)
            return carry

        ntri = (c1 - c0 + 2) // 3
        lax.fori_loop(0, ntri, tri_body, 0)
        pltpu.sync_copy(canvas, part_hbm.at[wid])


_scatter_sc = functools.partial(
    pl.kernel,
    out_type=jax.ShapeDtypeStruct((_NWORK, _HW), jnp.float32),
    mesh=plsc.VectorSubcoreMesh(core_axis_name="c", subcore_axis_name="s"),
    compiler_params=pltpu.CompilerParams(needs_layout_passes=False),
    scratch_types=[
        pltpu.VMEM((_HW,), jnp.float32),
        pltpu.VMEM((_PP, _CHUNK), jnp.float32),
        pltpu.VMEM((_PP, _CHUNK), jnp.float32),
        pltpu.VMEM((_PP, _CHUNK), jnp.float32),
        pltpu.VMEM((_QB,), jnp.int32),
        pltpu.SemaphoreType.DMA,
        pltpu.SemaphoreType.DMA,
        pltpu.SemaphoreType.DMA,
    ],
)(_sc_body)


def _reduce_body(p_ref, o_ref):
    o_ref[...] = jnp.sum(p_ref[...], axis=1)


def kernel(patches, qstart):
    pt = jnp.transpose(patches.reshape(_T, _N, _ROW), (0, 2, 1))
    partials = _scatter_sc(pt.reshape(_T, _ROW, 1, _N), qstart)
    p4 = partials.reshape(_NCOMBO, _SUBS, _HW // 128, 128)
    vid6 = pl.pallas_call(
        _reduce_body,
        grid=(_NCOMBO,),
        in_specs=[
            pl.BlockSpec((1, _SUBS, _HW // 128, 128), lambda i: (i, 0, 0, 0))
        ],
        out_specs=pl.BlockSpec((1, _HW // 128, 128), lambda i: (i, 0, 0)),
        out_shape=jax.ShapeDtypeStruct((_NCOMBO, _HW // 128, 128), jnp.float32),
    )(p4)
    return vid6.reshape(_T, _C, _H, _W)
